# baseline (speedup 1.0000x reference)
import functools

import jax
import jax.numpy as jnp
from jax import lax
from jax.experimental import pallas as pl
from jax.experimental.pallas import tpu as pltpu
from jax.experimental.pallas import tpu_sc as plsc

_NC = 2
_NS = 16
_NW = _NC * _NS
_D = 32
_NROWS = 1000000
_LAST = 999936
_RING = 4


def _gmf_t(user_idx, item_idx, utab_t, itab_t):
    batch = user_idx.shape[0]
    bpw = batch // _NW
    ngrp = bpw // 16
    mesh = plsc.VectorSubcoreMesh(core_axis_name="c", subcore_axis_name="s")

    import dataclasses
    cp = pltpu.CompilerParams()
    if "needs_layout_passes" in pltpu.CompilerParams.__dataclass_fields__:
        cp = dataclasses.replace(cp, needs_layout_passes=False)

    @functools.partial(
        pl.kernel,
        mesh=mesh,
        compiler_params=cp,
        out_type=jax.ShapeDtypeStruct((_D, batch), jnp.float32),
        scratch_types=(
            [pltpu.VMEM((bpw + 16,), jnp.int32),
             pltpu.VMEM((bpw + 16,), jnp.int32)]
            + [pltpu.VMEM((_D, 128), jnp.float32) for _ in range(2 * _RING)]
            + [pltpu.VMEM((_D, 64), jnp.float32),
               pltpu.VMEM((_D, 64), jnp.float32),
               pltpu.VMEM((_D, bpw), jnp.float32)]
            + [pltpu.SemaphoreType.DMA for _ in range(2 * _RING + 1)]
        ),
    )
    def k(uidx_hbm, iidx_hbm, utab_hbm, itab_hbm, out_hbm, uv, iv, *rest):
        bufs_u = rest[0:_RING]
        bufs_i = rest[_RING:2 * _RING]
        tail_u, tail_i, stage = rest[2 * _RING:2 * _RING + 3]
        sems_u = rest[2 * _RING + 3:3 * _RING + 3]
        sems_i = rest[3 * _RING + 3:4 * _RING + 3]
        sem = rest[4 * _RING + 3]

        wid = lax.axis_index("s") * _NC + lax.axis_index("c")
        base = wid * bpw
        cp0 = pltpu.async_copy(uidx_hbm.at[pl.ds(base, bpw)],
                               uv.at[pl.ds(0, bpw)], sem)
        cp1 = pltpu.async_copy(iidx_hbm.at[pl.ds(base, bpw)],
                               iv.at[pl.ds(0, bpw)], sem)
        cp2 = pltpu.async_copy(utab_hbm.at[:, pl.ds(_LAST, 64)], tail_u, sem)
        cp3 = pltpu.async_copy(itab_hbm.at[:, pl.ds(_LAST, 64)], tail_i, sem)
        for c in (cp0, cp1, cp2, cp3):
            c.wait()

        jvec = lax.iota(jnp.int32, 16)

        def fire(u, i, slot):
            tcu = jnp.minimum(u >> 7, 7811) << 7
            tci = jnp.minimum(i >> 7, 7811) << 7
            pltpu.async_copy(
                utab_hbm.at[:, pl.ds(pl.multiple_of(tcu, 128), 128)],
                bufs_u[slot], sems_u[slot])
            pltpu.async_copy(
                itab_hbm.at[:, pl.ds(pl.multiple_of(tci, 128), 128)],
                bufs_i[slot], sems_i[slot])

        def wait_slot(slot):
            pltpu.make_async_copy(utab_hbm.at[:, pl.ds(0, 128)],
                                  bufs_u[slot], sems_u[slot]).wait()
            pltpu.make_async_copy(itab_hbm.at[:, pl.ds(0, 128)],
                                  bufs_i[slot], sems_i[slot]).wait()

        def column(u, buf, tail):
            lane = u & 127
            is_tail = u >= _LAST
            tlane = jnp.where(is_tail, u - _LAST, 0)
            halves = []
            for h in range(2):
                rows = jvec + h * 16
                vmain = plsc.load_gather(buf, [rows, lane + jvec * 0])
                vtail = plsc.load_gather(tail, [rows, tlane + jvec * 0])
                halves.append(jnp.where(is_tail, vtail, vmain))
            return halves

        vec0_u = uv[pl.ds(0, 16)]
        vec0_i = iv[pl.ds(0, 16)]
        for p in range(_RING):
            fire(vec0_u[p], vec0_i[p], p)

        @pl.loop(0, ngrp)
        def _(g):
            cur_u = uv[pl.ds(g * 16, 16)]
            cur_i = iv[pl.ds(g * 16, 16)]
            nxt_u = uv[pl.ds(g * 16 + 16, 16)]
            nxt_i = iv[pl.ds(g * 16 + 16, 16)]
            for kk in range(16):
                b = g * 16 + kk
                slot = kk % _RING
                wait_slot(slot)
                hu = column(cur_u[kk], bufs_u[slot], tail_u)
                hi = column(cur_i[kk], bufs_i[slot], tail_i)
                for h in range(2):
                    plsc.store_scatter(stage, [jvec + h * 16, b + jvec * 0],
                                       hu[h] * hi[h])
                if kk + _RING < 16:
                    un, vn = cur_u[kk + _RING], cur_i[kk + _RING]
                else:
                    un, vn = nxt_u[kk + _RING - 16], nxt_i[kk + _RING - 16]

                @pl.when(jnp.logical_and(b + _RING < bpw, True))
                def _():
                    fire(un, vn, slot)

        pltpu.sync_copy(stage, out_hbm.at[:, pl.ds(base, bpw)])

    return k(user_idx, item_idx, utab_t, itab_t)


def kernel(user_input, item_input, user_table, item_table):
    out_t = _gmf_t(user_input.astype(jnp.int32), item_input.astype(jnp.int32),
                   user_table.T, item_table.T)
    return out_t.T


# speedup vs baseline: 1.0090x; 1.0090x over previous
import functools

import jax
import jax.numpy as jnp
from jax import lax
from jax.experimental import pallas as pl
from jax.experimental.pallas import tpu as pltpu
from jax.experimental.pallas import tpu_sc as plsc

_NC = 2
_NS = 16
_NW = _NC * _NS
_D = 32
_NROWS = 1000000
_LAST = 999936
_RING = 8


def _gmf_t(user_idx, item_idx, utab_t, itab_t):
    batch = user_idx.shape[0]
    bpw = batch // _NW
    ngrp = bpw // 16
    mesh = plsc.VectorSubcoreMesh(core_axis_name="c", subcore_axis_name="s")

    import dataclasses
    cp = pltpu.CompilerParams()
    if "needs_layout_passes" in pltpu.CompilerParams.__dataclass_fields__:
        cp = dataclasses.replace(cp, needs_layout_passes=False)

    @functools.partial(
        pl.kernel,
        mesh=mesh,
        compiler_params=cp,
        out_type=jax.ShapeDtypeStruct((_D, batch), jnp.float32),
        scratch_types=(
            [pltpu.VMEM((bpw + 16,), jnp.int32),
             pltpu.VMEM((bpw + 16,), jnp.int32)]
            + [pltpu.VMEM((_D, 128), jnp.float32) for _ in range(2 * _RING)]
            + [pltpu.VMEM((_D, 64), jnp.float32),
               pltpu.VMEM((_D, 64), jnp.float32),
               pltpu.VMEM((_D, bpw), jnp.float32)]
            + [pltpu.SemaphoreType.DMA for _ in range(2 * _RING + 1)]
        ),
    )
    def k(uidx_hbm, iidx_hbm, utab_hbm, itab_hbm, out_hbm, uv, iv, *rest):
        bufs_u = rest[0:_RING]
        bufs_i = rest[_RING:2 * _RING]
        tail_u, tail_i, stage = rest[2 * _RING:2 * _RING + 3]
        sems_u = rest[2 * _RING + 3:3 * _RING + 3]
        sems_i = rest[3 * _RING + 3:4 * _RING + 3]
        sem = rest[4 * _RING + 3]

        wid = lax.axis_index("s") * _NC + lax.axis_index("c")
        base = wid * bpw
        cp0 = pltpu.async_copy(uidx_hbm.at[pl.ds(base, bpw)],
                               uv.at[pl.ds(0, bpw)], sem)
        cp1 = pltpu.async_copy(iidx_hbm.at[pl.ds(base, bpw)],
                               iv.at[pl.ds(0, bpw)], sem)
        cp2 = pltpu.async_copy(utab_hbm.at[:, pl.ds(_LAST, 64)], tail_u, sem)
        cp3 = pltpu.async_copy(itab_hbm.at[:, pl.ds(_LAST, 64)], tail_i, sem)
        for c in (cp0, cp1, cp2, cp3):
            c.wait()

        jvec = lax.iota(jnp.int32, 16)

        def fire(u, i, slot):
            tcu = jnp.minimum(u >> 7, 7811) << 7
            tci = jnp.minimum(i >> 7, 7811) << 7
            pltpu.async_copy(
                utab_hbm.at[:, pl.ds(pl.multiple_of(tcu, 128), 128)],
                bufs_u[slot], sems_u[slot])
            pltpu.async_copy(
                itab_hbm.at[:, pl.ds(pl.multiple_of(tci, 128), 128)],
                bufs_i[slot], sems_i[slot])

        def wait_slot(slot):
            pltpu.make_async_copy(utab_hbm.at[:, pl.ds(0, 128)],
                                  bufs_u[slot], sems_u[slot]).wait()
            pltpu.make_async_copy(itab_hbm.at[:, pl.ds(0, 128)],
                                  bufs_i[slot], sems_i[slot]).wait()

        def column(u, buf, tail):
            lane = u & 127
            is_tail = u >= _LAST
            tlane = jnp.where(is_tail, u - _LAST, 0)
            halves = []
            for h in range(2):
                rows = jvec + h * 16
                vmain = plsc.load_gather(buf, [rows, lane + jvec * 0])
                vtail = plsc.load_gather(tail, [rows, tlane + jvec * 0])
                halves.append(jnp.where(is_tail, vtail, vmain))
            return halves

        vec0_u = uv[pl.ds(0, 16)]
        vec0_i = iv[pl.ds(0, 16)]
        for p in range(_RING):
            fire(vec0_u[p], vec0_i[p], p)

        @pl.loop(0, ngrp)
        def _(g):
            cur_u = uv[pl.ds(g * 16, 16)]
            cur_i = iv[pl.ds(g * 16, 16)]
            nxt_u = uv[pl.ds(g * 16 + 16, 16)]
            nxt_i = iv[pl.ds(g * 16 + 16, 16)]
            for kk in range(16):
                b = g * 16 + kk
                slot = kk % _RING
                wait_slot(slot)
                hu = column(cur_u[kk], bufs_u[slot], tail_u)
                hi = column(cur_i[kk], bufs_i[slot], tail_i)
                for h in range(2):
                    plsc.store_scatter(stage, [jvec + h * 16, b + jvec * 0],
                                       hu[h] * hi[h])
                if kk + _RING < 16:
                    un, vn = cur_u[kk + _RING], cur_i[kk + _RING]
                else:
                    un, vn = nxt_u[kk + _RING - 16], nxt_i[kk + _RING - 16]

                @pl.when(jnp.logical_and(b + _RING < bpw, True))
                def _():
                    fire(un, vn, slot)

        pltpu.sync_copy(stage, out_hbm.at[:, pl.ds(base, bpw)])

    return k(user_idx, item_idx, utab_t, itab_t)


def kernel(user_input, item_input, user_table, item_table):
    out_t = _gmf_t(user_input.astype(jnp.int32), item_input.astype(jnp.int32),
                   user_table.T, item_table.T)
    return out_t.T
